# baseline (speedup 1.0000x reference)
import functools

import jax
import jax.numpy as jnp
from jax import lax
from jax.experimental import pallas as pl
from jax.experimental.pallas import tpu as pltpu
from jax.experimental.pallas import tpu_sc as plsc

K_NN = 32
FEATURES = 128
HEADS = 4
_NSUB = 16


def _sc_gather(tab, idx):
    B, HN, E = tab.shape
    _, N, HK = idx.shape
    per = N // _NSUB
    mesh = plsc.VectorSubcoreMesh(core_axis_name="c", subcore_axis_name="s")

    @functools.partial(
        pl.kernel,
        mesh=mesh,
        out_type=jax.ShapeDtypeStruct((B, N, HK, E), jnp.float32),
        scratch_types=[
            pltpu.VMEM((per, HK), jnp.int32),
            pltpu.VMEM((2, HK, E), jnp.float32),
            pltpu.SemaphoreType.DMA,
            pltpu.SemaphoreType.DMA,
            pltpu.SemaphoreType.DMA,
            pltpu.SemaphoreType.DMA,
        ],
    )
    def k(tab_hbm, idx_hbm, out_hbm, idx_v, rows_v, sg0, sg1, sw0, sw1):
        b = lax.axis_index("c")
        s = lax.axis_index("s")
        base = s * per
        pltpu.sync_copy(idx_hbm.at[b, pl.ds(base, per)], idx_v)
        gsems = (sg0, sg1)
        wsems = (sw0, sw1)

        @pl.loop(0, per, step=2)
        def _(g):
            for u in range(2):
                n = g + u
                @pl.when(n >= 2)
                def _():
                    pltpu.make_async_copy(
                        rows_v.at[u], out_hbm.at[b, base + n - 2], wsems[u]
                    ).wait()
                pltpu.async_copy(
                    tab_hbm.at[b].at[idx_v.at[n]], rows_v.at[u], gsems[u]
                ).wait()
                pltpu.async_copy(rows_v.at[u], out_hbm.at[b, base + n], wsems[u])

        for u in range(2):
            pltpu.make_async_copy(
                rows_v.at[u], out_hbm.at[b, base + per - 2 + u], wsems[u]
            ).wait()

    return k(tab, idx)


def _proj_body(pc_ref, W1_ref, b1_ref, W2_ref, b2_ref, Wk1_ref, bk1_ref,
               tab_ref, pT_ref):
    x = pc_ref[0]
    for h in range(HEADS):
        proj_h = jax.nn.relu(
            jnp.dot(x, Wk1_ref[h], preferred_element_type=jnp.float32)
            + bk1_ref[h][None, :])
        tab_ref[0, h] = proj_h
        p1 = jax.nn.relu(
            jnp.dot(x, W1_ref[h], preferred_element_type=jnp.float32)
            + b1_ref[h][None, :])
        p2 = jax.nn.relu(
            jnp.dot(p1, W2_ref[h], preferred_element_type=jnp.float32)
            + b2_ref[h][None, :])
        pT_ref[0, :, h] = p2[:, 0]


def _proj(point_cloud, W1, b1, W2, b2, Wk1, bk1):
    B, N, F = point_cloud.shape
    blk = 512
    return pl.pallas_call(
        _proj_body,
        grid=(B, N // blk),
        in_specs=[
            pl.BlockSpec((1, blk, F), lambda b, i: (b, i, 0)),
            pl.BlockSpec(W1.shape, lambda b, i: (0, 0, 0)),
            pl.BlockSpec(b1.shape, lambda b, i: (0, 0)),
            pl.BlockSpec(W2.shape, lambda b, i: (0, 0, 0)),
            pl.BlockSpec(b2.shape, lambda b, i: (0, 0)),
            pl.BlockSpec(Wk1.shape, lambda b, i: (0, 0, 0)),
            pl.BlockSpec(bk1.shape, lambda b, i: (0, 0)),
        ],
        out_specs=[
            pl.BlockSpec((1, HEADS, blk, FEATURES), lambda b, i: (b, 0, i, 0)),
            pl.BlockSpec((1, blk, HEADS), lambda b, i: (b, i, 0)),
        ],
        out_shape=[
            jax.ShapeDtypeStruct((B, HEADS, N, FEATURES), jnp.float32),
            jax.ShapeDtypeStruct((B, N, HEADS), jnp.float32),
        ],
    )(point_cloud, W1, b1, W2, b2, Wk1, bk1)


def _lex_less(ad, ai, bd, bi):
    return (ad < bd) | ((ad == bd) & (ai < bi))


def _ce(vd, vi, j, k_dir):
    M, L = vd.shape
    G = M // (2 * j)
    vd4 = vd.reshape(G, 2, j, L)
    vi4 = vi.reshape(G, 2, j, L)
    ad, bd = vd4[:, 0], vd4[:, 1]
    ai, bi = vi4[:, 0], vi4[:, 1]
    take_a = _lex_less(ad, ai, bd, bi)
    lo_d = jnp.where(take_a, ad, bd)
    hi_d = jnp.where(take_a, bd, ad)
    lo_i = jnp.where(take_a, ai, bi)
    hi_i = jnp.where(take_a, bi, ai)
    if k_dir is None:
        o0d, o1d, o0i, o1i = lo_d, hi_d, lo_i, hi_i
    else:
        g = jax.lax.broadcasted_iota(jnp.int32, ad.shape, 0)
        dirv = ((g * (2 * j)) & k_dir) == 0
        o0d = jnp.where(dirv, lo_d, hi_d)
        o1d = jnp.where(dirv, hi_d, lo_d)
        o0i = jnp.where(dirv, lo_i, hi_i)
        o1i = jnp.where(dirv, hi_i, lo_i)
    vd = jnp.stack([o0d, o1d], axis=1).reshape(M, L)
    vi = jnp.stack([o0i, o1i], axis=1).reshape(M, L)
    return vd, vi


def _pair_min(vd, vi, half):
    M, L = vd.shape
    G = M // (2 * half)
    vd4 = vd.reshape(G, 2, half, L)
    vi4 = vi.reshape(G, 2, half, L)
    ad, bd = vd4[:, 0], vd4[:, 1]
    ai, bi = vi4[:, 0], vi4[:, 1]
    take_a = _lex_less(ad, ai, bd, bi)
    return (jnp.where(take_a, ad, bd).reshape(M // 2, L),
            jnp.where(take_a, ai, bi).reshape(M // 2, L))


def _topk_body(pc_ref, pcT_ref, idxT_ref):
    pc = pc_ref[0]
    pct = pcT_ref[0]
    N = pc.shape[0]
    R = pct.shape[1]
    inner = -2.0 * jnp.dot(pc, pct, preferred_element_type=jnp.float32)
    sq_all = jnp.sum(pc * pc, axis=1, keepdims=True)
    sq_blk = jnp.sum(pct * pct, axis=0, keepdims=True)
    vd = (sq_blk + inner) + sq_all
    vi = jax.lax.broadcasted_iota(jnp.int32, (N, R), 0)

    big = jnp.float32(jnp.inf)
    for q in range(K_NN):
        m = jnp.min(vd, axis=0)
        sel = vd == m[None, :]
        istar = jnp.min(jnp.where(sel, vi, N), axis=0)
        idxT_ref[0, q, :] = istar
        vd = jnp.where(sel & (vi == istar[None, :]), big, vd)


def _topk(point_cloud, pcT):
    B, N, F = point_cloud.shape
    R = 128
    return pl.pallas_call(
        _topk_body,
        grid=(B, N // R),
        in_specs=[
            pl.BlockSpec((1, N, F), lambda b, i: (b, 0, 0)),
            pl.BlockSpec((1, F, R), lambda b, i: (b, 0, i)),
        ],
        out_specs=pl.BlockSpec((1, K_NN, R), lambda b, i: (b, 0, i)),
        out_shape=jax.ShapeDtypeStruct((B, K_NN, N), jnp.int32),
    )(point_cloud, pcT)


def _att_body(gf_ref, pT_ref, Wk2_ref, bk2_ref, att_ref, coeff_ref):
    x = gf_ref[0]
    R = x.shape[0]
    att = []
    coeff = []
    for h in range(HEADS):
        xh = x[:, h].reshape(R * K_NN, FEATURES)
        m2 = jax.nn.relu(
            jnp.dot(xh, Wk2_ref[h], preferred_element_type=jnp.float32)
            + bk2_ref[h][None, :])
        logits = pT_ref[0, :, h][:, None] + m2.reshape(R, K_NN)
        logits = jnp.where(logits > 0, logits, 0.3 * logits)
        mx = jnp.max(logits, axis=-1, keepdims=True)
        e = jnp.exp(logits - mx)
        c = e / jnp.sum(e, axis=-1, keepdims=True)
        coeff.append(c)
        att.append(jnp.sum(c[:, :, None] * x[:, h], axis=1))
    att_ref[0] = jnp.stack(att, axis=1)
    coeff_ref[0] = jnp.stack(coeff, axis=1)


def _att_coeff(gf, pT, Wk2, bk2):
    B, N = gf.shape[:2]
    R = 64
    return pl.pallas_call(
        _att_body,
        grid=(B, N // R),
        in_specs=[
            pl.BlockSpec((1, R, HEADS, K_NN, FEATURES), lambda b, i: (b, i, 0, 0, 0)),
            pl.BlockSpec((1, R, HEADS), lambda b, i: (b, i, 0)),
            pl.BlockSpec(Wk2.shape, lambda b, i: (0, 0, 0)),
            pl.BlockSpec(bk2.shape, lambda b, i: (0, 0)),
        ],
        out_specs=[
            pl.BlockSpec((1, R, HEADS, FEATURES), lambda b, i: (b, i, 0, 0)),
            pl.BlockSpec((1, R, HEADS, K_NN), lambda b, i: (b, i, 0, 0)),
        ],
        out_shape=[
            jax.ShapeDtypeStruct((B, N, HEADS, FEATURES), jnp.float32),
            jax.ShapeDtypeStruct((B, N, HEADS, K_NN), jnp.float32),
        ],
    )(gf, pT, Wk2, bk2)


def kernel(point_cloud, W1, b1, W2, b2, Wk1, bk1, Wk2, bk2):
    B, N, F = point_cloud.shape
    tab, pT = _proj(point_cloud, W1, b1, W2, b2, Wk1, bk1)
    idxT = _topk(point_cloud, jnp.swapaxes(point_cloud, 1, 2))
    nn_idx = jnp.swapaxes(idxT, 1, 2)

    idx_ex = (nn_idx[:, :, None, :]
              + (jnp.arange(HEADS, dtype=nn_idx.dtype) * N)[None, None, :, None])
    gf = _sc_gather(tab.reshape(B, HEADS * N, FEATURES),
                    idx_ex.reshape(B, N, HEADS * K_NN))
    gf = gf.reshape(B, N, HEADS, K_NN, FEATURES)

    att, coeff = _att_coeff(gf, pT, Wk2, bk2)
    return (att, gf, coeff)


# speedup vs baseline: 1.0882x; 1.0882x over previous
import functools

import jax
import jax.numpy as jnp
from jax import lax
from jax.experimental import pallas as pl
from jax.experimental.pallas import tpu as pltpu
from jax.experimental.pallas import tpu_sc as plsc

K_NN = 32
FEATURES = 128
HEADS = 4
_NSUB = 16


def _sc_gather(tab, idx):
    B, HN, E = tab.shape
    _, N, HK = idx.shape
    per = N // _NSUB
    mesh = plsc.VectorSubcoreMesh(core_axis_name="c", subcore_axis_name="s")

    @functools.partial(
        pl.kernel,
        mesh=mesh,
        out_type=jax.ShapeDtypeStruct((B, N, HK, E), jnp.float32),
        scratch_types=[
            pltpu.VMEM((per, HK), jnp.int32),
            pltpu.VMEM((2, HK, E), jnp.float32),
            pltpu.SemaphoreType.DMA,
            pltpu.SemaphoreType.DMA,
            pltpu.SemaphoreType.DMA,
            pltpu.SemaphoreType.DMA,
        ],
    )
    def k(tab_hbm, idx_hbm, out_hbm, idx_v, rows_v, sg0, sg1, sw0, sw1):
        b = lax.axis_index("c")
        s = lax.axis_index("s")
        base = s * per
        pltpu.sync_copy(idx_hbm.at[b, pl.ds(base, per)], idx_v)
        gsems = (sg0, sg1)
        wsems = (sw0, sw1)

        @pl.loop(0, per, step=2)
        def _(g):
            for u in range(2):
                n = g + u
                @pl.when(n >= 2)
                def _():
                    pltpu.make_async_copy(
                        rows_v.at[u], out_hbm.at[b, base + n - 2], wsems[u]
                    ).wait()
                pltpu.async_copy(
                    tab_hbm.at[b].at[idx_v.at[n]], rows_v.at[u], gsems[u]
                ).wait()
                pltpu.async_copy(rows_v.at[u], out_hbm.at[b, base + n], wsems[u])

        for u in range(2):
            pltpu.make_async_copy(
                rows_v.at[u], out_hbm.at[b, base + per - 2 + u], wsems[u]
            ).wait()

    return k(tab, idx)


def _proj_body(pc_ref, W1_ref, b1_ref, W2_ref, b2_ref, Wk1_ref, bk1_ref,
               tab_ref, pT_ref):
    x = pc_ref[0]
    for h in range(HEADS):
        proj_h = jax.nn.relu(
            jnp.dot(x, Wk1_ref[h], preferred_element_type=jnp.float32)
            + bk1_ref[h][None, :])
        tab_ref[0, h] = proj_h
        p1 = jax.nn.relu(
            jnp.dot(x, W1_ref[h], preferred_element_type=jnp.float32)
            + b1_ref[h][None, :])
        p2 = jax.nn.relu(
            jnp.dot(p1, W2_ref[h], preferred_element_type=jnp.float32)
            + b2_ref[h][None, :])
        pT_ref[0, :, h] = p2[:, 0]


def _proj(point_cloud, W1, b1, W2, b2, Wk1, bk1):
    B, N, F = point_cloud.shape
    blk = 512
    return pl.pallas_call(
        _proj_body,
        grid=(B, N // blk),
        in_specs=[
            pl.BlockSpec((1, blk, F), lambda b, i: (b, i, 0)),
            pl.BlockSpec(W1.shape, lambda b, i: (0, 0, 0)),
            pl.BlockSpec(b1.shape, lambda b, i: (0, 0)),
            pl.BlockSpec(W2.shape, lambda b, i: (0, 0, 0)),
            pl.BlockSpec(b2.shape, lambda b, i: (0, 0)),
            pl.BlockSpec(Wk1.shape, lambda b, i: (0, 0, 0)),
            pl.BlockSpec(bk1.shape, lambda b, i: (0, 0)),
        ],
        out_specs=[
            pl.BlockSpec((1, HEADS, blk, FEATURES), lambda b, i: (b, 0, i, 0)),
            pl.BlockSpec((1, blk, HEADS), lambda b, i: (b, i, 0)),
        ],
        out_shape=[
            jax.ShapeDtypeStruct((B, HEADS, N, FEATURES), jnp.float32),
            jax.ShapeDtypeStruct((B, N, HEADS), jnp.float32),
        ],
    )(point_cloud, W1, b1, W2, b2, Wk1, bk1)


def _adjw_body(pcb_ref, pcT_ref, adj_ref):
    x = pcb_ref[0]
    xt = pcT_ref[0]
    inner = -2.0 * jnp.dot(x, xt, preferred_element_type=jnp.float32)
    sq = jnp.sum(x * x, axis=1, keepdims=True)
    sqT = jnp.sum(xt * xt, axis=0, keepdims=True)
    adj_ref[0] = (sq + inner) + sqT


def _adjw(point_cloud, pcT):
    B, N, F = point_cloud.shape
    blk = 512
    return pl.pallas_call(
        _adjw_body,
        grid=(B, N // blk),
        in_specs=[
            pl.BlockSpec((1, blk, F), lambda b, i: (b, i, 0)),
            pl.BlockSpec((1, F, N), lambda b, i: (b, 0, 0)),
        ],
        out_specs=pl.BlockSpec((1, blk, N), lambda b, i: (b, i, 0)),
        out_shape=jax.ShapeDtypeStruct((B, N, N), jnp.float32),
    )(point_cloud, pcT)


def _topk_body(adj_ref, idx_ref):
    vd = adj_ref[0]
    R, N = vd.shape
    vi = jax.lax.broadcasted_iota(jnp.int32, (R, N), 1)

    big = jnp.float32(jnp.inf)
    for q in range(K_NN):
        m = jnp.min(vd, axis=1, keepdims=True)
        sel = vd == m
        istar = jnp.min(jnp.where(sel, vi, N), axis=1, keepdims=True)
        idx_ref[0, :, q] = istar[:, 0]
        vd = jnp.where(sel & (vi == istar), big, vd)


def _topk(adj):
    B, N, _ = adj.shape
    R = 128
    return pl.pallas_call(
        _topk_body,
        grid=(B, N // R),
        in_specs=[
            pl.BlockSpec((1, R, N), lambda b, i: (b, i, 0)),
        ],
        out_specs=pl.BlockSpec((1, R, K_NN), lambda b, i: (b, i, 0)),
        out_shape=jax.ShapeDtypeStruct((B, N, K_NN), jnp.int32),
    )(adj)


def _att_body(gf_ref, pT_ref, Wk2_ref, bk2_ref, att_ref, coeff_ref):
    x = gf_ref[0]
    R = x.shape[0]
    att = []
    coeff = []
    for h in range(HEADS):
        xh = x[:, h].reshape(R * K_NN, FEATURES)
        m2 = jax.nn.relu(
            jnp.dot(xh, Wk2_ref[h], preferred_element_type=jnp.float32)
            + bk2_ref[h][None, :])
        logits = pT_ref[0, :, h][:, None] + m2.reshape(R, K_NN)
        logits = jnp.where(logits > 0, logits, 0.3 * logits)
        mx = jnp.max(logits, axis=-1, keepdims=True)
        e = jnp.exp(logits - mx)
        c = e / jnp.sum(e, axis=-1, keepdims=True)
        coeff.append(c)
        att.append(jnp.sum(c[:, :, None] * x[:, h], axis=1))
    att_ref[0] = jnp.stack(att, axis=1)
    coeff_ref[0] = jnp.stack(coeff, axis=1)


def _att_coeff(gf, pT, Wk2, bk2):
    B, N = gf.shape[:2]
    R = 64
    return pl.pallas_call(
        _att_body,
        grid=(B, N // R),
        in_specs=[
            pl.BlockSpec((1, R, HEADS, K_NN, FEATURES), lambda b, i: (b, i, 0, 0, 0)),
            pl.BlockSpec((1, R, HEADS), lambda b, i: (b, i, 0)),
            pl.BlockSpec(Wk2.shape, lambda b, i: (0, 0, 0)),
            pl.BlockSpec(bk2.shape, lambda b, i: (0, 0)),
        ],
        out_specs=[
            pl.BlockSpec((1, R, HEADS, FEATURES), lambda b, i: (b, i, 0, 0)),
            pl.BlockSpec((1, R, HEADS, K_NN), lambda b, i: (b, i, 0, 0)),
        ],
        out_shape=[
            jax.ShapeDtypeStruct((B, N, HEADS, FEATURES), jnp.float32),
            jax.ShapeDtypeStruct((B, N, HEADS, K_NN), jnp.float32),
        ],
    )(gf, pT, Wk2, bk2)


def kernel(point_cloud, W1, b1, W2, b2, Wk1, bk1, Wk2, bk2):
    B, N, F = point_cloud.shape
    tab, pT = _proj(point_cloud, W1, b1, W2, b2, Wk1, bk1)
    pcT = jax.lax.optimization_barrier(jnp.swapaxes(point_cloud, 1, 2))
    adj = jax.lax.optimization_barrier(_adjw(point_cloud, pcT))
    nn_idx = _topk(adj)

    idx_ex = (nn_idx[:, :, None, :]
              + (jnp.arange(HEADS, dtype=nn_idx.dtype) * N)[None, None, :, None])
    gf = _sc_gather(tab.reshape(B, HEADS * N, FEATURES),
                    idx_ex.reshape(B, N, HEADS * K_NN))
    gf = gf.reshape(B, N, HEADS, K_NN, FEATURES)

    att, coeff = _att_coeff(gf, pT, Wk2, bk2)
    return (att, gf, coeff)
